# baseline (device time: 18794 ns/iter reference)
import jax
import jax.numpy as jnp
from jax import lax
from jax.experimental import pallas as pl
from jax.experimental.pallas import tpu as pltpu


def kernel(x):
    m, n = x.shape
    half = n // 2
    hrows = m // 2

    def body(x_ref, out_ref, local_sem, a_send, a_recv, b_send, b_recv):
        my_x = lax.axis_index("x")
        my_y = lax.axis_index("y")
        my_z = lax.axis_index("z")
        peer_y = 1 - my_y
        peer_x = 1 - my_x

        y_peer = (my_x, peer_y, my_z)
        diag = (peer_x, peer_y, my_z)

        barrier_sem = pltpu.get_barrier_semaphore()
        for dev in (y_peer, diag):
            pl.semaphore_signal(
                barrier_sem, inc=1,
                device_id=dev, device_id_type=pl.DeviceIdType.MESH,
            )
        pl.semaphore_wait(barrier_sem, 2)

        src = x_ref.at[pl.ds(my_x * hrows, hrows), pl.ds(peer_y * half, half)]
        dst_rows = my_y * m + my_x * hrows

        a = pltpu.make_async_remote_copy(
            src_ref=src,
            dst_ref=out_ref.at[pl.ds(dst_rows, hrows), :],
            send_sem=a_send, recv_sem=a_recv,
            device_id=y_peer, device_id_type=pl.DeviceIdType.MESH,
        )
        b = pltpu.make_async_remote_copy(
            src_ref=src,
            dst_ref=out_ref.at[pl.ds(dst_rows, hrows), :],
            send_sem=b_send, recv_sem=b_recv,
            device_id=diag, device_id_type=pl.DeviceIdType.MESH,
        )
        a.start()
        b.start()

        copy = pltpu.make_async_copy(
            x_ref.at[:, pl.ds(my_y * half, half)],
            out_ref.at[pl.ds(my_y * m, m), :],
            local_sem,
        )
        copy.start()

        a.wait()
        b.wait()
        copy.wait()

    return pl.pallas_call(
        body,
        out_shape=jax.ShapeDtypeStruct((2 * m, half), x.dtype),
        in_specs=[pl.BlockSpec(memory_space=pl.ANY)],
        out_specs=pl.BlockSpec(memory_space=pl.ANY),
        scratch_shapes=[
            pltpu.SemaphoreType.DMA,
            pltpu.SemaphoreType.DMA,
            pltpu.SemaphoreType.DMA,
            pltpu.SemaphoreType.DMA,
            pltpu.SemaphoreType.DMA,
        ],
        compiler_params=pltpu.CompilerParams(collective_id=0),
    )(x)


# device time: 14533 ns/iter; 1.2932x vs baseline; 1.2932x over previous
import jax
import jax.numpy as jnp
from jax import lax
from jax.experimental import pallas as pl
from jax.experimental.pallas import tpu as pltpu

FCH = 3
FC = 64
D0 = 192
DN = 128


def kernel(x):
    m, n = x.shape
    half = n // 2

    def body(x_ref, out_ref, local_sem,
             yf_send, yf_recv, yd_send, yd_recv, f_send, f_recv):
        my_x = lax.axis_index("x")
        my_y = lax.axis_index("y")
        my_z = lax.axis_index("z")
        peer_y = 1 - my_y
        peer_x = 1 - my_x

        y_peer = (my_x, peer_y, my_z)
        x_peer = (peer_x, my_y, my_z)

        barrier_sem = pltpu.get_barrier_semaphore()
        for dev in (y_peer, x_peer):
            pl.semaphore_signal(
                barrier_sem, inc=1,
                device_id=dev, device_id_type=pl.DeviceIdType.MESH,
            )
        pl.semaphore_wait(barrier_sem, 2)

        copy = pltpu.make_async_copy(
            x_ref.at[:, pl.ds(my_y * half, half)],
            out_ref.at[pl.ds(my_y * m, m), :],
            local_sem,
        )
        copy.start()

        fs = my_x * (FCH * FC + DN)
        cols = pl.ds(peer_y * half, half)

        yf = []
        for k in range(FCH):
            r = fs + k * FC
            rd = pltpu.make_async_remote_copy(
                src_ref=x_ref.at[pl.ds(r, FC), cols],
                dst_ref=out_ref.at[pl.ds(my_y * m + r, FC), :],
                send_sem=yf_send.at[k], recv_sem=yf_recv.at[k],
                device_id=y_peer, device_id_type=pl.DeviceIdType.MESH,
            )
            rd.start()
            yf.append(rd)
        yd = pltpu.make_async_remote_copy(
            src_ref=x_ref.at[pl.ds(D0, DN), cols],
            dst_ref=out_ref.at[pl.ds(my_y * m + D0, DN), :],
            send_sem=yd_send, recv_sem=yd_recv,
            device_id=y_peer, device_id_type=pl.DeviceIdType.MESH,
        )
        yd.start()

        fwds = []
        for k in range(FCH):
            yf[k].wait_recv()
            rows = pl.ds(peer_y * m + fs + k * FC, FC)
            fwd = pltpu.make_async_remote_copy(
                src_ref=out_ref.at[rows, :],
                dst_ref=out_ref.at[rows, :],
                send_sem=f_send.at[k], recv_sem=f_recv.at[k],
                device_id=x_peer, device_id_type=pl.DeviceIdType.MESH,
            )
            fwd.start()
            fwds.append(fwd)

        ps = peer_x * (FCH * FC + DN)
        for k in range(FCH):
            prcv = pltpu.make_async_remote_copy(
                src_ref=x_ref.at[pl.ds(0, FC), pl.ds(0, half)],
                dst_ref=out_ref.at[pl.ds(peer_y * m + ps + k * FC, FC), :],
                send_sem=f_send.at[k],
                recv_sem=f_recv.at[k],
                device_id=x_peer, device_id_type=pl.DeviceIdType.MESH,
            )
            prcv.wait_recv()

        yd.wait()
        for k in range(FCH):
            yf[k].wait_send()
            fwds[k].wait_send()
        copy.wait()

    return pl.pallas_call(
        body,
        out_shape=jax.ShapeDtypeStruct((2 * m, half), x.dtype),
        in_specs=[pl.BlockSpec(memory_space=pl.ANY)],
        out_specs=pl.BlockSpec(memory_space=pl.ANY),
        scratch_shapes=[
            pltpu.SemaphoreType.DMA,
            pltpu.SemaphoreType.DMA((FCH,)),
            pltpu.SemaphoreType.DMA((FCH,)),
            pltpu.SemaphoreType.DMA,
            pltpu.SemaphoreType.DMA,
            pltpu.SemaphoreType.DMA((FCH,)),
            pltpu.SemaphoreType.DMA((FCH,)),
        ],
        compiler_params=pltpu.CompilerParams(collective_id=0),
    )(x)
